# Initial kernel scaffold; baseline (speedup 1.0000x reference)
import jax
import jax.numpy as jnp
from jax.experimental import pallas as pl


def kernel(superposition_weights, W):
    raise NotImplementedError("write your pallas kernel here")


# speedup vs baseline: 1.5318x; 1.5318x over previous
import jax
import jax.numpy as jnp
from jax.experimental import pallas as pl


def _mm_block(sw_ref, w_ref, o_ref):
    o_ref[...] = jax.lax.dot_general(
        sw_ref[...], w_ref[...],
        dimension_numbers=(((0,), (0,)), ((), ())),
        preferred_element_type=jnp.float32,
    )


def kernel(superposition_weights, W):
    E, B = superposition_weights.shape
    _, d1, d2 = W.shape
    N = d1 * d2
    W2 = W.reshape(E, N)
    blk = 8192
    out = pl.pallas_call(
        _mm_block,
        grid=(N // blk,),
        in_specs=[
            pl.BlockSpec((E, B), lambda i: (0, 0)),
            pl.BlockSpec((E, blk), lambda i: (0, i)),
        ],
        out_specs=pl.BlockSpec((B, blk), lambda i: (0, i)),
        out_shape=jax.ShapeDtypeStruct((B, N), jnp.float32),
    )(superposition_weights, W2)
    return out.reshape(B, d1, d2)


# speedup vs baseline: 4.4957x; 2.9349x over previous
import jax
import jax.numpy as jnp
from jax.experimental import pallas as pl

_BD1 = 16


def _mm_block(sw_ref, w_ref, o_ref):
    sw = sw_ref[...]
    for k in range(_BD1):
        o_ref[:, k, :] = jax.lax.dot_general(
            sw, w_ref[:, k, :],
            dimension_numbers=(((0,), (0,)), ((), ())),
            preferred_element_type=jnp.float32,
        )


def kernel(superposition_weights, W):
    E, B = superposition_weights.shape
    _, d1, d2 = W.shape
    return pl.pallas_call(
        _mm_block,
        grid=(d1 // _BD1,),
        in_specs=[
            pl.BlockSpec((E, B), lambda i: (0, 0)),
            pl.BlockSpec((E, _BD1, d2), lambda i: (0, i, 0)),
        ],
        out_specs=pl.BlockSpec((B, _BD1, d2), lambda i: (0, i, 0)),
        out_shape=jax.ShapeDtypeStruct((B, d1, d2), jnp.float32),
    )(superposition_weights, W)
